# baseline (device time: 41627 ns/iter reference)
import jax
import jax.numpy as jnp
from jax import lax
from jax.experimental import pallas as pl
from jax.experimental.pallas import tpu as pltpu

N_DEV = 4
N_LOCAL_EXPERTS = 4
N_RINGS = 4
RING_DIRS = (+1, +1, -1, -1)


def kernel(x, router_W, route_idx, expert_W):
    n_tok, d_model = x.shape
    n_experts = router_W.shape[1]
    d_out = expert_W.shape[2]
    chunk = d_out // N_DEV
    qrows = n_tok // N_RINGS

    def body(x_ref, rw_ref, idx_ref, ew_ref, out_ref, *scr):
        send_bufs = scr[0:N_RINGS]
        recv_bufs = scr[N_RINGS:2 * N_RINGS]
        ag_bufs = scr[2 * N_RINGS:3 * N_RINGS]
        sem = scr[3 * N_RINGS:]
        rs_ss = sem[0:N_RINGS]
        rs_rs = sem[N_RINGS:2 * N_RINGS]
        ag_ss = sem[2 * N_RINGS:3 * N_RINGS]
        ag_rs = sem[3 * N_RINGS:4 * N_RINGS]

        my = lax.axis_index("i")
        left = lax.rem(my + N_DEV - 1, N_DEV)
        right = lax.rem(my + 1, N_DEV)

        barrier_sem = pltpu.get_barrier_semaphore()
        for nbr in (left, right):
            pl.semaphore_signal(
                barrier_sem, inc=1,
                device_id=(nbr,), device_id_type=pl.DeviceIdType.MESH,
            )
        pl.semaphore_wait(barrier_sem, 2)

        def gates_block(k):
            r0 = k * qrows
            xf = x_ref[r0:r0 + qrows, :]
            scores = lax.dot_general(
                xf.astype(jnp.bfloat16), rw_ref[:, :].astype(jnp.bfloat16),
                (((1,), (0,)), ((), ())),
                preferred_element_type=jnp.float32,
            )
            s_max = jnp.max(scores, axis=1, keepdims=True)
            probs = jnp.exp(scores - s_max)
            probs = probs / jnp.sum(probs, axis=1, keepdims=True)

            e0 = idx_ref[r0:r0 + qrows, 0:1]
            e1 = idx_ref[r0:r0 + qrows, 1:2]
            expert_iota = lax.broadcasted_iota(
                jnp.int32, (qrows, n_experts), 1)
            g0 = jnp.sum(
                jnp.where(e0 == expert_iota, probs, 0.0),
                axis=1, keepdims=True)
            g1 = jnp.sum(
                jnp.where(e1 == expert_iota, probs, 0.0),
                axis=1, keepdims=True)
            gs = g0 + g1
            xwb = []
            for j in range(N_LOCAL_EXPERTS):
                gid = my * N_LOCAL_EXPERTS + j
                w_j = (
                    (g0 / gs) * (e0 == gid).astype(jnp.float32)
                    + (g1 / gs) * (e1 == gid).astype(jnp.float32)
                )
                xwb.append((xf * w_j).astype(jnp.bfloat16))
            return xwb

        xw_blocks = [None] * N_RINGS

        def pquarter(c, k):
            if xw_blocks[k] is None:
                xw_blocks[k] = gates_block(k)
            acc = jnp.zeros((qrows, chunk), dtype=jnp.float32)
            for j in range(N_LOCAL_EXPERTS):
                W_j = ew_ref[j, :, pl.ds(c * chunk, chunk)].astype(jnp.bfloat16)
                acc = acc + lax.dot_general(
                    xw_blocks[k][j], W_j, (((1,), (0,)), ((), ())),
                    preferred_element_type=jnp.float32,
                )
            return acc

        def copy(src, dst, ss, rs, dev):
            return pltpu.make_async_remote_copy(
                src_ref=src, dst_ref=dst, send_sem=ss, recv_sem=rs,
                device_id=(dev,), device_id_type=pl.DeviceIdType.MESH,
            )

        def rs_chunk(k, s):
            if RING_DIRS[k] > 0:
                return lax.rem(my + N_DEV - 1 - s, N_DEV)
            return lax.rem(my + 1 + s, N_DEV)

        dests = [right if d > 0 else left for d in RING_DIRS]

        rdmas = [None] * N_RINGS
        for k in range(N_RINGS):
            send_bufs[k][0, :, :] = pquarter(my, k).astype(jnp.bfloat16)
            rdmas[k] = copy(send_bufs[k].at[0], recv_bufs[k].at[0],
                            rs_ss[k].at[0], rs_rs[k].at[0], dests[k])
            rdmas[k].start()
        accs = [None] * N_RINGS
        for s in range(N_DEV - 1):
            for k in range(N_RINGS):
                nxt = pquarter(rs_chunk(k, s), k)
                rdmas[k].wait()
                accs[k] = recv_bufs[k][s, :, :].astype(jnp.float32) + nxt
                if s < N_DEV - 2:
                    send_bufs[k][s + 1, :, :] = accs[k].astype(jnp.bfloat16)
                    rdmas[k] = copy(
                        send_bufs[k].at[s + 1], recv_bufs[k].at[s + 1],
                        rs_ss[k].at[s + 1], rs_rs[k].at[s + 1], dests[k])
                    rdmas[k].start()

        def own_chunk(k, p):
            if RING_DIRS[k] > 0:
                return lax.rem(p + 1, N_DEV)
            return lax.rem(p + N_DEV - 1, N_DEV)

        sends = []
        for k in range(N_RINGS):
            ag_bufs[k][0, :, :] = accs[k].astype(jnp.bfloat16)
            for r in range(1, N_DEV):
                dest = lax.rem(my + r, N_DEV)
                rho = N_DEV - r
                sk = copy(ag_bufs[k].at[0], ag_bufs[k].at[rho],
                          ag_ss[k].at[r - 1], ag_rs[k].at[rho - 1], dest)
                sk.start()
                sends.append(sk)
        for k in range(N_RINGS):
            out_ref[k * qrows:(k + 1) * qrows,
                    pl.ds(own_chunk(k, my) * chunk, chunk)] = (
                ag_bufs[k][0, :, :]
            )

        for rho in (1, 3, 2):
            p = lax.rem(my + rho, N_DEV)
            for k in range(N_RINGS):
                rk = copy(ag_bufs[k].at[0], ag_bufs[k].at[rho],
                          ag_ss[k].at[0], ag_rs[k].at[rho - 1], my)
                rk.wait_recv()
                out_ref[k * qrows:(k + 1) * qrows,
                        pl.ds(own_chunk(k, p) * chunk, chunk)] = (
                    ag_bufs[k][rho, :, :]
                )
        for sk in sends:
            sk.wait_send()

    qbuf = lambda n: pltpu.VMEM((n, qrows, chunk), jnp.bfloat16)
    sems = lambda: pltpu.SemaphoreType.DMA((N_DEV - 1,))
    scratch = (
        [qbuf(N_DEV - 1) for _ in range(N_RINGS)]
        + [qbuf(N_DEV - 1) for _ in range(N_RINGS)]
        + [qbuf(N_DEV) for _ in range(N_RINGS)]
        + [sems() for _ in range(4 * N_RINGS)]
    )
    return pl.pallas_call(
        body,
        out_shape=jax.ShapeDtypeStruct((n_tok, d_out), jnp.bfloat16),
        in_specs=[
            pl.BlockSpec(memory_space=pltpu.VMEM),
            pl.BlockSpec(memory_space=pltpu.VMEM),
            pl.BlockSpec(memory_space=pltpu.VMEM),
            pl.BlockSpec(memory_space=pltpu.VMEM),
        ],
        out_specs=pl.BlockSpec(memory_space=pltpu.VMEM),
        scratch_shapes=scratch,
        compiler_params=pltpu.CompilerParams(collective_id=0),
    )(x, router_W, route_idx, expert_W)
